# baseline (speedup 1.0000x reference)
import dataclasses
import functools

import jax
import jax.numpy as jnp
from jax import lax
from jax.experimental import pallas as pl
from jax.experimental.pallas import tpu as pltpu
from jax.experimental.pallas import tpu_sc as plsc

B = 16384
EMB = 32
NC = 2
NS = 16
NW = NC * NS
BPW = B // NW

_HI = jax.lax.Precision.HIGHEST

_SC_PARAMS = pltpu.CompilerParams(use_tc_tiling_on_sc=False)


def _sc_gather(mlp_u, mlp_i, gmf_u, gmf_i, users, items):
    mesh = plsc.VectorSubcoreMesh(core_axis_name="c", subcore_axis_name="s")
    row = pltpu.MemorySpace.HBM((B, EMB), jnp.float32)

    @functools.partial(
        pl.kernel,
        mesh=mesh,
        out_type=[row, row, row, row],
        compiler_params=_SC_PARAMS,
        scratch_types=[
            pltpu.VMEM((BPW,), jnp.int32),
            pltpu.VMEM((BPW,), jnp.int32),
            pltpu.VMEM((BPW, EMB), jnp.float32),
            pltpu.VMEM((BPW, EMB), jnp.float32),
            pltpu.VMEM((BPW, EMB), jnp.float32),
            pltpu.VMEM((BPW, EMB), jnp.float32),
            pltpu.SemaphoreType.DMA,
        ],
    )
    def k(mu_hbm, mi_hbm, gu_hbm, gi_hbm, u_hbm, it_hbm,
          omu, omi, ogu, ogi,
          uidx, iidx, mu_v, mi_v, gu_v, gi_v, sem):
        wid = lax.axis_index("s") * NC + lax.axis_index("c")
        base = wid * BPW
        pltpu.sync_copy(u_hbm.at[pl.ds(base, BPW)], uidx)
        pltpu.sync_copy(it_hbm.at[pl.ds(base, BPW)], iidx)
        c1 = pltpu.async_copy(mu_hbm.at[uidx], mu_v, sem)
        c2 = pltpu.async_copy(mi_hbm.at[iidx], mi_v, sem)
        c3 = pltpu.async_copy(gu_hbm.at[uidx], gu_v, sem)
        c4 = pltpu.async_copy(gi_hbm.at[iidx], gi_v, sem)
        c1.wait()
        c2.wait()
        c3.wait()
        c4.wait()
        pltpu.sync_copy(mu_v, omu.at[pl.ds(base, BPW)])
        pltpu.sync_copy(mi_v, omi.at[pl.ds(base, BPW)])
        pltpu.sync_copy(gu_v, ogu.at[pl.ds(base, BPW)])
        pltpu.sync_copy(gi_v, ogi.at[pl.ds(base, BPW)])

    return k(mlp_u, mlp_i, gmf_u, gmf_i, users, items)


def _tc_body(mu, mi, gu, gi, w0a, w0b, b0, w1, b1, w2, b2, wm, wg, bias, out):
    h = jnp.dot(mu[...], w0a[...], precision=_HI) + jnp.dot(mi[...], w0b[...], precision=_HI)
    h = jnp.maximum(h + b0[...], 0.0)
    h = jnp.maximum(jnp.dot(h, w1[...], precision=_HI) + b1[...], 0.0)
    h = jnp.maximum(jnp.dot(h, w2[...], precision=_HI) + b2[...], 0.0)
    y = jnp.sum(h * wm[...], axis=1)
    y = y + jnp.sum((gu[...] * gi[...]) * wg[...], axis=1)
    out[...] = (y + bias[0, 0])[:, None]


def _tc_dense(mu, mi, gu, gi, w0a, w0b, b0, w1, b1, w2, b2, wm, wg, bias):
    blk = 2048
    grid = (B // blk,)
    emb_spec = pl.BlockSpec((blk, EMB), lambda i: (i, 0))

    def full(shape):
        return pl.BlockSpec(shape, lambda i: (0,) * len(shape))

    return pl.pallas_call(
        _tc_body,
        grid=grid,
        in_specs=[
            emb_spec, emb_spec, emb_spec, emb_spec,
            full((EMB, 128)), full((EMB, 128)), full((1, 128)),
            full((128, 64)), full((1, 64)),
            full((64, 32)), full((1, 32)),
            full((1, 32)), full((1, 32)), full((1, 1)),
        ],
        out_specs=pl.BlockSpec((blk, 1), lambda i: (i, 0)),
        out_shape=jax.ShapeDtypeStruct((B, 1), jnp.float32),
    )(mu, mi, gu, gi, w0a, w0b, b0, w1, b1, w2, b2, wm, wg, bias)


def kernel(users, items, mlp_user_table, mlp_item_table, gmf_user_table,
           gmf_item_table, mlp_W0, mlp_b0, mlp_W1, mlp_b1, mlp_W2, mlp_b2,
           mlp_fc_w, mlp_fc_b, gmf_fc_w, gmf_fc_b):
    users = users.astype(jnp.int32)
    items = items.astype(jnp.int32)
    mu, mi, gu, gi = _sc_gather(mlp_user_table, mlp_item_table,
                                gmf_user_table, gmf_item_table, users, items)
    w0a = mlp_W0[:EMB]
    w0b = mlp_W0[EMB:]
    bias = (mlp_fc_b + gmf_fc_b).reshape(1, 1)
    y = _tc_dense(mu, mi, gu, gi,
                  w0a, w0b, mlp_b0.reshape(1, -1),
                  mlp_W1, mlp_b1.reshape(1, -1),
                  mlp_W2, mlp_b2.reshape(1, -1),
                  mlp_fc_w.reshape(1, -1), gmf_fc_w.reshape(1, -1), bias)
    return y[:, 0]


# speedup vs baseline: 12.0250x; 12.0250x over previous
import jax
import jax.numpy as jnp
from jax import lax
from jax.experimental import pallas as pl
from jax.experimental.pallas import tpu as pltpu

B = 16384
EMB = 32
NC = 2
NS = 16
NW = NC * NS
BPW = B // NW
VL = 16

_HI = jax.lax.Precision.HIGHEST


def _tc_body(mu_r, mi_r, gu_r, gi_r, w0a, w0b, b0, w1, b1, w2, b2, wm, wg, bias, out):
    mu, mi, gu, gi = mu_r[...], mi_r[...], gu_r[...], gi_r[...]
    h = jnp.dot(w0a[...], mu, precision=_HI) + jnp.dot(w0b[...], mi, precision=_HI)
    h = jnp.maximum(h + b0[...], 0.0)
    h = jnp.maximum(jnp.dot(w1[...], h, precision=_HI) + b1[...], 0.0)
    h = jnp.maximum(jnp.dot(w2[...], h, precision=_HI) + b2[...], 0.0)
    y = jnp.sum(h * wm[...], axis=0)
    y = y + jnp.sum((gu * gi) * wg[...], axis=0)
    out[...] = (y + bias[0, 0])[None, :]


def _tc_dense(mu_t, mi_t, gu_t, gi_t, w0a, w0b, b0, w1, b1, w2, b2, wm, wg, bias):
    blk = 2048
    grid = (B // blk,)
    emb_spec = pl.BlockSpec((EMB, blk), lambda i: (0, i))

    def full(shape):
        return pl.BlockSpec(shape, lambda i: (0,) * len(shape))

    return pl.pallas_call(
        _tc_body,
        grid=grid,
        in_specs=[
            emb_spec, emb_spec, emb_spec, emb_spec,
            full((128, EMB)), full((128, EMB)), full((128, 1)),
            full((64, 128)), full((64, 1)),
            full((EMB, 64)), full((EMB, 1)),
            full((EMB, 1)), full((EMB, 1)), full((1, 1)),
        ],
        out_specs=pl.BlockSpec((1, blk), lambda i: (0, i)),
        out_shape=jax.ShapeDtypeStruct((1, B), jnp.float32),
    )(mu_t, mi_t, gu_t, gi_t, w0a, w0b, b0, w1, b1, w2, b2, wm, wg, bias)


def kernel(users, items, mlp_user_table, mlp_item_table, gmf_user_table,
           gmf_item_table, mlp_W0, mlp_b0, mlp_W1, mlp_b1, mlp_W2, mlp_b2,
           mlp_fc_w, mlp_fc_b, gmf_fc_w, gmf_fc_b):
    users = users.astype(jnp.int32)
    items = items.astype(jnp.int32)
    mu_t = jnp.take(mlp_user_table.T, users, axis=1)
    mi_t = jnp.take(mlp_item_table.T, items, axis=1)
    gu_t = jnp.take(gmf_user_table.T, users, axis=1)
    gi_t = jnp.take(gmf_item_table.T, items, axis=1)
    w0a = mlp_W0[:EMB].T
    w0b = mlp_W0[EMB:].T
    bias = (mlp_fc_b + gmf_fc_b).reshape(1, 1)
    y = _tc_dense(mu_t, mi_t, gu_t, gi_t,
                  w0a, w0b, mlp_b0.reshape(-1, 1),
                  mlp_W1.T, mlp_b1.reshape(-1, 1),
                  mlp_W2.T, mlp_b2.reshape(-1, 1),
                  mlp_fc_w.reshape(-1, 1), gmf_fc_w.reshape(-1, 1), bias)
    return y[0]


# speedup vs baseline: 14.0190x; 1.1658x over previous
import jax
import jax.numpy as jnp
from jax import lax
from jax.experimental import pallas as pl
from jax.experimental.pallas import tpu as pltpu

B = 16384
EMB = 32
NC = 2
NS = 16
NW = NC * NS
BPW = B // NW
VL = 16

_HI = jax.lax.Precision.HIGHEST


def _tc_body(mu_r, mi_r, gu_r, gi_r, w0, b0, w1, b1, w2, b2, wm, wg, bias, out):
    gu, gi = gu_r[...], gi_r[...]
    x = jnp.concatenate([mu_r[...], mi_r[...]], axis=0)
    h = jnp.maximum(jnp.dot(w0[...], x) + b0[...], 0.0)
    h = jnp.maximum(jnp.dot(w1[...], h) + b1[...], 0.0)
    h = jnp.maximum(jnp.dot(w2[...], h) + b2[...], 0.0)
    y = jnp.sum(h * wm[...], axis=0)
    y = y + jnp.sum((gu * gi) * wg[...], axis=0)
    out[...] = (y + bias[0, 0])[None, :]


def _tc_dense(mu_t, mi_t, gu_t, gi_t, w0, b0, w1, b1, w2, b2, wm, wg, bias):
    blk = 2048
    grid = (B // blk,)
    emb_spec = pl.BlockSpec((EMB, blk), lambda i: (0, i))

    def full(shape):
        return pl.BlockSpec(shape, lambda i: (0,) * len(shape))

    return pl.pallas_call(
        _tc_body,
        grid=grid,
        in_specs=[
            emb_spec, emb_spec, emb_spec, emb_spec,
            full((128, 2 * EMB)), full((128, 1)),
            full((64, 128)), full((64, 1)),
            full((EMB, 64)), full((EMB, 1)),
            full((EMB, 1)), full((EMB, 1)), full((1, 1)),
        ],
        out_specs=pl.BlockSpec((1, blk), lambda i: (0, i)),
        out_shape=jax.ShapeDtypeStruct((1, B), jnp.float32),
    )(mu_t, mi_t, gu_t, gi_t, w0, b0, w1, b1, w2, b2, wm, wg, bias)


def kernel(users, items, mlp_user_table, mlp_item_table, gmf_user_table,
           gmf_item_table, mlp_W0, mlp_b0, mlp_W1, mlp_b1, mlp_W2, mlp_b2,
           mlp_fc_w, mlp_fc_b, gmf_fc_w, gmf_fc_b):
    users = users.astype(jnp.int32)
    items = items.astype(jnp.int32)
    _m = "clip"
    mu_t = jnp.take(mlp_user_table.T, users, axis=1, mode=_m)
    mi_t = jnp.take(mlp_item_table.T, items, axis=1, mode=_m)
    gu_t = jnp.take(gmf_user_table.T, users, axis=1, mode=_m)
    gi_t = jnp.take(gmf_item_table.T, items, axis=1, mode=_m)
    bias = (mlp_fc_b + gmf_fc_b).reshape(1, 1)
    y = _tc_dense(mu_t, mi_t, gu_t, gi_t,
                  mlp_W0.T, mlp_b0.reshape(-1, 1),
                  mlp_W1.T, mlp_b1.reshape(-1, 1),
                  mlp_W2.T, mlp_b2.reshape(-1, 1),
                  mlp_fc_w.reshape(-1, 1), gmf_fc_w.reshape(-1, 1), bias)
    return y[0]


# speedup vs baseline: 14.3199x; 1.0215x over previous
import jax
import jax.numpy as jnp
from jax import lax
from jax.experimental import pallas as pl
from jax.experimental.pallas import tpu as pltpu

B = 16384
EMB = 32
NC = 2
NS = 16
NW = NC * NS
BPW = B // NW
VL = 16

_HI = jax.lax.Precision.HIGHEST


def _mlp_body(mu_r, mi_r, w0, b0, w1, b1, w2, b2, wm, out):
    x = jnp.concatenate([mu_r[...], mi_r[...]], axis=0)
    h = jnp.maximum(jnp.dot(w0[...], x) + b0[...], 0.0)
    h = jnp.maximum(jnp.dot(w1[...], h) + b1[...], 0.0)
    h = jnp.maximum(jnp.dot(w2[...], h) + b2[...], 0.0)
    out[...] = jnp.sum(h * wm[...], axis=0)[None, :]


def _gmf_body(gu_r, gi_r, wg, ym, bias, out):
    y = jnp.sum((gu_r[...] * gi_r[...]) * wg[...], axis=0)
    out[...] = ym[...] + (y + bias[0, 0])[None, :]


_BLK = 2048


def _emb_spec():
    return pl.BlockSpec((EMB, _BLK), lambda i: (0, i))


def _full(shape):
    return pl.BlockSpec(shape, lambda i: (0,) * len(shape))


def _tc_mlp(mu_t, mi_t, w0, b0, w1, b1, w2, b2, wm):
    return pl.pallas_call(
        _mlp_body,
        grid=(B // _BLK,),
        in_specs=[
            _emb_spec(), _emb_spec(),
            _full((128, 2 * EMB)), _full((128, 1)),
            _full((64, 128)), _full((64, 1)),
            _full((EMB, 64)), _full((EMB, 1)),
            _full((EMB, 1)),
        ],
        out_specs=pl.BlockSpec((1, _BLK), lambda i: (0, i)),
        out_shape=jax.ShapeDtypeStruct((1, B), jnp.float32),
    )(mu_t, mi_t, w0, b0, w1, b1, w2, b2, wm)


def _tc_gmf(gu_t, gi_t, wg, ym, bias):
    return pl.pallas_call(
        _gmf_body,
        grid=(B // _BLK,),
        in_specs=[
            _emb_spec(), _emb_spec(),
            _full((EMB, 1)),
            pl.BlockSpec((1, _BLK), lambda i: (0, i)),
            _full((1, 1)),
        ],
        out_specs=pl.BlockSpec((1, _BLK), lambda i: (0, i)),
        out_shape=jax.ShapeDtypeStruct((1, B), jnp.float32),
    )(gu_t, gi_t, wg, ym, bias)


def kernel(users, items, mlp_user_table, mlp_item_table, gmf_user_table,
           gmf_item_table, mlp_W0, mlp_b0, mlp_W1, mlp_b1, mlp_W2, mlp_b2,
           mlp_fc_w, mlp_fc_b, gmf_fc_w, gmf_fc_b):
    users = users.astype(jnp.int32)
    items = items.astype(jnp.int32)
    _m = "clip"
    mu_t = jnp.take(mlp_user_table.T, users, axis=1, mode=_m)
    mi_t = jnp.take(mlp_item_table.T, items, axis=1, mode=_m)
    gu_t = jnp.take(gmf_user_table.T, users, axis=1, mode=_m)
    gi_t = jnp.take(gmf_item_table.T, items, axis=1, mode=_m)
    bias = (mlp_fc_b + gmf_fc_b).reshape(1, 1)
    ym = _tc_mlp(mu_t, mi_t,
                 mlp_W0.T, mlp_b0.reshape(-1, 1),
                 mlp_W1.T, mlp_b1.reshape(-1, 1),
                 mlp_W2.T, mlp_b2.reshape(-1, 1),
                 mlp_fc_w.reshape(-1, 1))
    y = _tc_gmf(gu_t, gi_t, gmf_fc_w.reshape(-1, 1), ym, bias)
    return y[0]
